# Initial kernel scaffold; baseline (speedup 1.0000x reference)
import jax
import jax.numpy as jnp
from jax.experimental import pallas as pl


def kernel(x, edge_index, W_gat, attn_l, attn_r, b_gat, W1, b1, W2, b2):
    raise NotImplementedError("write your pallas kernel here")


# speedup vs baseline: 36.2342x; 36.2342x over previous
import jax
import jax.numpy as jnp
from jax import lax
from jax.experimental import pallas as pl
from jax.experimental.pallas import tpu as pltpu
from jax.experimental.pallas import tpu_sc as plsc

N = 10000
E = 320000
D = 128
H = 2

_HIGH = lax.Precision.HIGHEST


_BLK_A = 1000


def _feat_body(x_ref, w_ref, al_ref, ar_ref, feat_ref, el_ref, er_ref):
    blk = jnp.dot(x_ref[...], w_ref[...], precision=_HIGH,
                  preferred_element_type=jnp.float32)
    f0 = blk[:, :D]
    f1 = blk[:, D:]
    feat_ref[0] = f0
    feat_ref[1] = f1
    el_ref[...] = jnp.stack([jnp.sum(f0 * al_ref[0][None, :], axis=1),
                             jnp.sum(f1 * al_ref[1][None, :], axis=1)], axis=1)
    er_ref[...] = jnp.stack([jnp.sum(f0 * ar_ref[0][None, :], axis=1),
                             jnp.sum(f1 * ar_ref[1][None, :], axis=1)], axis=1)


def _run_feat(x, W_gat, attn_l, attn_r):
    return pl.pallas_call(
        _feat_body,
        grid=(N // _BLK_A,),
        in_specs=[
            pl.BlockSpec((_BLK_A, D), lambda i: (i, 0)),
            pl.BlockSpec((D, H * D), lambda i: (0, 0)),
            pl.BlockSpec((H, D), lambda i: (0, 0)),
            pl.BlockSpec((H, D), lambda i: (0, 0)),
        ],
        out_specs=[
            pl.BlockSpec((H, _BLK_A, D), lambda i: (0, i, 0)),
            pl.BlockSpec((_BLK_A, H), lambda i: (i, 0)),
            pl.BlockSpec((_BLK_A, H), lambda i: (i, 0)),
        ],
        out_shape=[
            jax.ShapeDtypeStruct((H, N, D), jnp.float32),
            jax.ShapeDtypeStruct((N, H), jnp.float32),
            jax.ShapeDtypeStruct((N, H), jnp.float32),
        ],
    )(x, W_gat, attn_l, attn_r)


_EPT = E // 16
_SB = 2000


def _s0_body(el_hbm, er_hbm, ei_hbm, w_hbm, esump_hbm,
             el_v, er_v, src_v, dst_v, w_st, esum_v):
    c = lax.axis_index("c")
    s = lax.axis_index("s")
    pltpu.sync_copy(el_hbm, el_v)
    pltpu.sync_copy(er_hbm, er_v)

    zero16 = jnp.zeros((16,), jnp.float32)

    def _zero(r, _):
        esum_v[pl.ds(r * 16, 16)] = zero16
        return _
    lax.fori_loop(0, N // 16, _zero, 0)

    base_e = s * _EPT
    lane = lax.iota(jnp.int32, 16)

    def _stage(st, _0):
        pltpu.sync_copy(ei_hbm.at[pl.ds(base_e + st * _SB, _SB)], src_v)
        pltpu.sync_copy(ei_hbm.at[pl.ds(E + base_e + st * _SB, _SB)], dst_v)

        def _grp(j, _):
            sidx = src_v[pl.ds(j * 16, 16)]
            didx = dst_v[pl.ds(j * 16, 16)]
            el16 = plsc.load_gather(el_v, [sidx * 2 + c])
            er16 = plsc.load_gather(er_v, [didx * 2 + c])
            e16 = el16 + er16
            e16 = jnp.where(e16 >= 0, e16, 0.2 * e16)
            w16 = jnp.exp(e16)
            w_st[pl.ds(j * 16, 16)] = w16
            for i in range(16):
                d = didx[i]
                ws = w16[i]
                b = d & ~15
                o = d & 15
                v = esum_v[pl.ds(b, 16)]
                esum_v[pl.ds(b, 16)] = v + jnp.where(lane == o, ws, 0.0)
            return _
        lax.fori_loop(0, _SB // 16, _grp, 0)
        pltpu.sync_copy(w_st, w_hbm.at[pl.ds(c * E + base_e + st * _SB, _SB)])
        return _0

    lax.fori_loop(0, _EPT // _SB, _stage, 0)
    pltpu.sync_copy(esum_v.at[pl.ds(0, N)],
                    esump_hbm.at[pl.ds((c * 16 + s) * N, N)])


def _run_s0(el, er, edge_index):
    mesh = plsc.VectorSubcoreMesh(core_axis_name="c", subcore_axis_name="s")
    f = pl.kernel(
        _s0_body,
        out_type=(
            jax.ShapeDtypeStruct((H * E,), jnp.float32),
            jax.ShapeDtypeStruct((H * 16 * N,), jnp.float32),
        ),
        mesh=mesh,
        compiler_params=pltpu.CompilerParams(needs_layout_passes=False),
        scratch_types=[
            pltpu.VMEM((H * N,), jnp.float32),
            pltpu.VMEM((H * N,), jnp.float32),
            pltpu.VMEM((_SB,), jnp.int32),
            pltpu.VMEM((_SB,), jnp.int32),
            pltpu.VMEM((_SB,), jnp.float32),
            pltpu.VMEM((N,), jnp.float32),
        ],
    )
    return f(el, er, edge_index)


_C1 = 80
_NCHUNK = _SB // _C1
_RB = 80
_NRCH = N // _RB


def _s1_body(feat_hbm, w_hbm, ei_hbm, acc_hbm,
             src_v, dst_v, w_sg, idx_v, dstidx_v, rows_v, acc_sp):
    c = lax.axis_index("c")
    s = lax.axis_index("s")
    base_e = s * _EPT

    zero16 = jnp.zeros((16,), jnp.float32)

    def _zero_row(r, _):
        for k in range(D // 16):
            rows_v[r, pl.ds(k * 16, 16)] = zero16
        return _
    lax.fori_loop(0, _C1, _zero_row, 0)
    for q in range(8):
        i = s + 16 * q

        @pl.when(i < _NRCH)
        def _():
            pltpu.sync_copy(rows_v, acc_sp.at[pl.ds(i * _RB, _RB)])
    plsc.subcore_barrier()

    cN = c * N

    def _stage(st, _0):
        pltpu.sync_copy(ei_hbm.at[pl.ds(base_e + st * _SB, _SB)], src_v)
        pltpu.sync_copy(ei_hbm.at[pl.ds(E + base_e + st * _SB, _SB)], dst_v)
        pltpu.sync_copy(w_hbm.at[pl.ds(c * E + base_e + st * _SB, _SB)], w_sg)

        def _chunk(k, _):
            cb = k * _C1
            for j in range(_C1 // 16):
                idx_v[pl.ds(j * 16, 16)] = src_v[pl.ds(cb + j * 16, 16)] + cN
                dstidx_v[pl.ds(j * 16, 16)] = dst_v[pl.ds(cb + j * 16, 16)]
            pltpu.sync_copy(feat_hbm.at[idx_v], rows_v)

            for j in range(_C1 // 16):
                w16 = w_sg[pl.ds(cb + j * 16, 16)]
                for i in range(16):
                    ws = w16[i]
                    row = j * 16 + i
                    for k2 in range(D // 16):
                        rows_v[row, pl.ds(k2 * 16, 16)] = (
                            rows_v[row, pl.ds(k2 * 16, 16)] * ws)

            pltpu.sync_copy(rows_v, acc_sp.at[dstidx_v], add=True)
            return _

        lax.fori_loop(0, _NCHUNK, _chunk, 0)
        return _0

    lax.fori_loop(0, _EPT // _SB, _stage, 0)
    plsc.subcore_barrier()

    for q in range(8):
        i = s + 16 * q

        @pl.when(i < _NRCH)
        def _():
            r = i * _RB
            pltpu.sync_copy(acc_sp.at[pl.ds(r, _RB)], rows_v)
            pltpu.sync_copy(rows_v, acc_hbm.at[pl.ds(cN + r, _RB)])


def _run_s1(feat, w, edge_index):
    mesh = plsc.VectorSubcoreMesh(core_axis_name="c", subcore_axis_name="s")
    f = pl.kernel(
        _s1_body,
        out_type=jax.ShapeDtypeStruct((H * N, D), jnp.float32),
        mesh=mesh,
        compiler_params=pltpu.CompilerParams(needs_layout_passes=False),
        scratch_types=[
            pltpu.VMEM((_SB,), jnp.int32),
            pltpu.VMEM((_SB,), jnp.int32),
            pltpu.VMEM((_SB,), jnp.float32),
            pltpu.VMEM((_C1,), jnp.int32),
            pltpu.VMEM((_C1,), jnp.int32),
            pltpu.VMEM((_C1, D), jnp.float32),
            pltpu.VMEM_SHARED((N, D), jnp.float32),
        ],
    )
    return f(feat, w, edge_index)


_BLK_B = 1000


def _pq_body(acc_ref, esump_ref, bg_ref, w1_ref, b1_ref, w2_ref, b2_ref,
             pq_ref):
    ep = esump_ref[...]
    s0 = jnp.sum(ep[:, :16], axis=1)[:, None]
    s1 = jnp.sum(ep[:, 16:], axis=1)[:, None]
    m0 = (jnp.where(s0 > 0, acc_ref[0] / s0, 0.0) + bg_ref[0, :D][None, :])
    m1 = (jnp.where(s1 > 0, acc_ref[1] / s1, 0.0) + bg_ref[0, D:][None, :])
    hm = jnp.maximum((m0 + m1) * 0.5, 0.0)
    u = jnp.dot(w1_ref[...], w2_ref[...], precision=_HIGH,
                preferred_element_type=jnp.float32)
    i = pl.program_id(0)
    sl = pl.ds(i * _BLK_B, _BLK_B)
    cterm = jnp.sum(b1_ref[0] * w2_ref[:, 0], dtype=jnp.float32) + b2_ref[0, 0]
    pq_ref[...] = jnp.stack(
        [jnp.sum(hm * u[:D, 0][None, :], axis=1) + cterm,
         jnp.sum(hm * u[D:, 0][None, :], axis=1)], axis=1)


def _run_pq(acc, esump, b_gat, W1, b1, W2, b2):
    return pl.pallas_call(
        _pq_body,
        grid=(N // _BLK_B,),
        in_specs=[
            pl.BlockSpec((H, _BLK_B, D), lambda i: (0, i, 0)),
            pl.BlockSpec((_BLK_B, H * 16), lambda i: (i, 0)),
            pl.BlockSpec((1, H * D), lambda i: (0, 0)),
            pl.BlockSpec((H * D, D), lambda i: (0, 0)),
            pl.BlockSpec((1, D), lambda i: (0, 0)),
            pl.BlockSpec((D, 1), lambda i: (0, 0)),
            pl.BlockSpec((1, 1), lambda i: (0, 0)),
        ],
        out_specs=pl.BlockSpec((_BLK_B, H), lambda i: (i, 0)),
        out_shape=jax.ShapeDtypeStruct((N, H), jnp.float32),
    )(acc.reshape(H, N, D), esump.reshape(H * 16, N).T,
      b_gat.reshape(1, H * D), W1, b1.reshape(1, D), W2, b2.reshape(1, 1))


_EPW = E // 32
_C2 = 2000


def _s2_body(pq_hbm, ei_hbm, out_hbm, pq_v, src_v, dst_v, o_v):
    c = lax.axis_index("c")
    s = lax.axis_index("s")
    wid = s * 2 + c
    pltpu.sync_copy(pq_hbm, pq_v)

    def _chunk(k, _):
        base = wid * _EPW + k * _C2
        pltpu.sync_copy(ei_hbm.at[pl.ds(base, _C2)], src_v)
        pltpu.sync_copy(ei_hbm.at[pl.ds(E + base, _C2)], dst_v)

        def _grp(j, _2):
            sidx = src_v[pl.ds(j * 16, 16)]
            didx = dst_v[pl.ds(j * 16, 16)]
            o_v[pl.ds(j * 16, 16)] = (plsc.load_gather(pq_v, [sidx * 2])
                                      + plsc.load_gather(pq_v, [didx * 2 + 1]))
            return _2
        lax.fori_loop(0, _C2 // 16, _grp, 0)
        pltpu.sync_copy(o_v, out_hbm.at[pl.ds(base, _C2)])
        return _

    lax.fori_loop(0, _EPW // _C2, _chunk, 0)


def _run_s2(pq, edge_index):
    mesh = plsc.VectorSubcoreMesh(core_axis_name="c", subcore_axis_name="s")
    f = pl.kernel(
        _s2_body,
        out_type=jax.ShapeDtypeStruct((E,), jnp.float32),
        mesh=mesh,
        compiler_params=pltpu.CompilerParams(needs_layout_passes=False),
        scratch_types=[
            pltpu.VMEM((H * N,), jnp.float32),
            pltpu.VMEM((_C2,), jnp.int32),
            pltpu.VMEM((_C2,), jnp.int32),
            pltpu.VMEM((_C2,), jnp.float32),
        ],
    )
    return f(pq, edge_index)


def kernel(x, edge_index, W_gat, attn_l, attn_r, b_gat, W1, b1, W2, b2):
    ei_flat = edge_index.reshape(2 * E)
    feat, el, er = _run_feat(x, W_gat, attn_l, attn_r)
    w, esump = _run_s0(el.reshape(H * N), er.reshape(H * N), ei_flat)
    acc = _run_s1(feat.reshape(H * N, D), w, ei_flat)
    pq = _run_pq(acc, esump, b_gat, W1, b1, W2, b2)
    score = _run_s2(pq.reshape(H * N), ei_flat)
    return score.reshape(E, 1)
